# baseline (speedup 1.0000x reference)
import math

import jax
import jax.numpy as jnp
from jax import lax
from jax.experimental import pallas as pl
from jax.experimental.pallas import tpu as pltpu
from jax.experimental.pallas import tpu_sc as plsc

VOCAB = 32000
N_TOK = 2048
PAD = 0
SMOOTHING = 0.1
CONF = 1.0 - SMOOTHING
EPS = SMOOTHING / (VOCAB - 2)
ROW_CONST = (VOCAB - 2) * EPS * math.log(EPS) + CONF * math.log(CONF)

RB = 256
CB = 3200

NW = 32
BPW = N_TOK // NW
LANES = 16
GATHER_SCALE = -(CONF - EPS)


def _dense_kernel(tgt_ref, x_ref, out_ref):
    i = pl.program_id(0)
    j = pl.program_id(1)

    @pl.when((i == 0) & (j == 0))
    def _():
        out_ref[...] = jnp.zeros((1, 1), jnp.float32)

    x = x_ref[...]
    tgt = tgt_ref[...]
    valid = tgt != PAD

    rowsum = jnp.sum(x, axis=1, keepdims=True)
    contrib = -EPS * rowsum
    contrib = contrib + jnp.where(j == 0, ROW_CONST + EPS * x[:, 0:1], 0.0)
    contrib = jnp.where(valid, contrib, 0.0)
    out_ref[...] += jnp.sum(contrib, axis=0, keepdims=True)


def _gather_body(x_hbm, tgt_hbm, out_hbm, idx_v, tgt_v, vals_v, acc_v, sem):
    wid = lax.axis_index("s") * 2 + lax.axis_index("c")
    base = wid * BPW
    pltpu.sync_copy(tgt_hbm.at[pl.ds(base, BPW)], tgt_v)
    for k in range(BPW // LANES):
        t16 = tgt_v[pl.ds(k * LANES, LANES)]
        rows = base + k * LANES + lax.iota(jnp.int32, LANES)
        idx_v[pl.ds(k * LANES, LANES)] = rows * VOCAB + t16
    pltpu.async_copy(x_hbm.at[idx_v], vals_v, sem).wait()
    acc = jnp.zeros((LANES,), jnp.float32)
    for k in range(BPW // LANES):
        t16 = tgt_v[pl.ds(k * LANES, LANES)]
        v16 = vals_v[pl.ds(k * LANES, LANES)]
        acc = acc + jnp.where(t16 != PAD, v16, 0.0)
    acc_v[...] = acc * GATHER_SCALE
    pltpu.sync_copy(acc_v, out_hbm.at[wid])


def _sc_gather(xflat, tgt):
    return pl.kernel(
        _gather_body,
        mesh=plsc.VectorSubcoreMesh(core_axis_name="c", subcore_axis_name="s"),
        out_type=jax.ShapeDtypeStruct((NW, LANES), jnp.float32),
        scratch_types=[
            pltpu.VMEM((BPW,), jnp.int32),
            pltpu.VMEM((BPW,), jnp.int32),
            pltpu.VMEM((BPW,), jnp.float32),
            pltpu.VMEM((LANES,), jnp.float32),
            pltpu.SemaphoreType.DMA,
        ],
    )(xflat, tgt)


@jax.jit
def kernel(x, target):
    tgt = target.astype(jnp.int32)
    partials = _sc_gather(x.reshape(N_TOK * VOCAB), tgt)
    dense = pl.pallas_call(
        _dense_kernel,
        grid=(N_TOK // RB, VOCAB // CB),
        in_specs=[
            pl.BlockSpec((RB, 1), lambda i, j: (i, 0)),
            pl.BlockSpec((RB, CB), lambda i, j: (i, j)),
        ],
        out_specs=pl.BlockSpec((1, 1), lambda i, j: (0, 0)),
        out_shape=jax.ShapeDtypeStruct((1, 1), jnp.float32),
        compiler_params=pltpu.CompilerParams(
            dimension_semantics=("arbitrary", "arbitrary"),
        ),
    )(tgt.reshape(N_TOK, 1), x)
    return dense[0, 0] + jnp.sum(partials)


# speedup vs baseline: 3.5718x; 3.5718x over previous
import math

import jax
import jax.numpy as jnp
from jax.experimental import pallas as pl
from jax.experimental.pallas import tpu as pltpu

VOCAB = 32000
N_TOK = 2048
PAD = 0
SMOOTHING = 0.1
CONF = 1.0 - SMOOTHING
EPS = SMOOTHING / (VOCAB - 2)
ROW_CONST = (VOCAB - 2) * EPS * math.log(EPS) + CONF * math.log(CONF)

RB = 128
CB = VOCAB


def _loss_kernel(tgt_ref, x_ref, out_ref):
    i = pl.program_id(0)

    @pl.when(i == 0)
    def _():
        out_ref[...] = jnp.zeros((1, 1), jnp.float32)

    x = x_ref[...]
    tgt = tgt_ref[...]
    valid = tgt != PAD

    rowsum = jnp.sum(x, axis=1, keepdims=True)
    cols = jax.lax.broadcasted_iota(jnp.int32, (RB, CB), 1)
    hit = cols == tgt
    xt = jnp.sum(jnp.where(hit, x, 0.0), axis=1, keepdims=True)

    contrib = ROW_CONST - EPS * rowsum + EPS * x[:, 0:1] - (CONF - EPS) * xt
    contrib = jnp.where(valid, contrib, 0.0)
    out_ref[...] += jnp.sum(contrib, axis=0, keepdims=True)


@jax.jit
def kernel(x, target):
    tgt = target.astype(jnp.int32).reshape(N_TOK, 1)
    out = pl.pallas_call(
        _loss_kernel,
        grid=(N_TOK // RB,),
        in_specs=[
            pl.BlockSpec((RB, 1), lambda i: (i, 0)),
            pl.BlockSpec((RB, CB), lambda i: (i, 0)),
        ],
        out_specs=pl.BlockSpec((1, 1), lambda i: (0, 0)),
        out_shape=jax.ShapeDtypeStruct((1, 1), jnp.float32),
        compiler_params=pltpu.CompilerParams(
            dimension_semantics=("arbitrary",),
        ),
    )(tgt, x)
    return out[0, 0]
